# Initial kernel scaffold; baseline (speedup 1.0000x reference)
import jax
import jax.numpy as jnp
from jax.experimental import pallas as pl


def kernel(x, edge_index, W_l, b_l, W_r, W1, b1, W2, b2):
    raise NotImplementedError("write your pallas kernel here")


# speedup vs baseline: 3.8994x; 3.8994x over previous
import jax
import jax.numpy as jnp
from jax import lax
from jax.experimental import pallas as pl
from jax.experimental.pallas import tpu as pltpu
from jax.experimental.pallas import tpu_sc as plsc

N = 10000
NP = 10240
E = 320000
D = 128
NC = 2
NS = 16
NW = NC * NS
CH = 128
NCHUNK = E // CH
ROWS_PER_TILE = NP // NS
ZR = 8
F32 = jnp.float32
I32 = jnp.int32


def _mesh():
    return plsc.VectorSubcoreMesh(core_axis_name="c", subcore_axis_name="s",
                                  num_cores=NC, num_subcores=NS)


def _worker(cid, sid):
    return sid * NC + cid


def _k1_body(src_hbm, dst_hbm, x_hbm, agg_out,
             sidx, didx, msg, agg_sh, zstage, sem):
    cid = lax.axis_index("c")
    sid = lax.axis_index("s")
    wid = _worker(cid, sid)
    row0 = sid * ROWS_PER_TILE

    zv = jnp.zeros((16,), F32)
    for i in range(ZR):
        for j in range(D // 16):
            zstage[i, pl.ds(j * 16, 16)] = zv
    for r in range(ROWS_PER_TILE // ZR):
        pltpu.sync_copy(zstage, agg_sh.at[pl.ds(row0 + r * ZR, ZR)])
    plsc.subcore_barrier()

    n_my = NCHUNK // NW + jnp.where(wid < NCHUNK % NW, 1, 0)

    def chunk(i, carry):
        base = (wid + i * NW) * CH
        pltpu.sync_copy(src_hbm.at[pl.ds(base, CH)], sidx)
        pltpu.sync_copy(dst_hbm.at[pl.ds(base, CH)], didx)
        pltpu.async_copy(x_hbm.at[sidx], msg, sem).wait()
        pltpu.sync_copy(msg, agg_sh.at[didx], add=True)
        return carry

    lax.fori_loop(0, n_my, chunk, 0)
    plsc.subcore_barrier()

    obase = cid * NP + row0
    for r in range(ROWS_PER_TILE // ZR):
        pltpu.sync_copy(agg_sh.at[pl.ds(row0 + r * ZR, ZR)], zstage)
        pltpu.sync_copy(zstage, agg_out.at[pl.ds(obase + r * ZR, ZR)])


def _k1(src, dst, x):
    f = pl.kernel(
        _k1_body,
        out_type=jax.ShapeDtypeStruct((NC * NP, D), F32),
        mesh=_mesh(),
        scratch_types=[
            pltpu.VMEM((CH,), I32),
            pltpu.VMEM((CH,), I32),
            pltpu.VMEM((CH, D), F32),
            pltpu.VMEM_SHARED((NP, D), F32),
            pltpu.VMEM((ZR, D), F32),
            pltpu.SemaphoreType.DMA,
        ],
    )
    return f(src, dst, x)


def _k1b_body(dst_hbm, cnt_out, didx, ones_v, cnt_sh, zstage):
    cid = lax.axis_index("c")
    sid = lax.axis_index("s")
    wid = _worker(cid, sid)
    row0 = sid * ROWS_PER_TILE

    zv = jnp.zeros((16,), F32)
    ov = jnp.full((16,), 1.0, F32)
    for i in range(ZR):
        for j in range(D // 16):
            zstage[i, pl.ds(j * 16, 16)] = zv
    for i in range(CH):
        ones_v[i, pl.ds(0, 16)] = ov
    for r in range(ROWS_PER_TILE // ZR):
        pltpu.sync_copy(zstage, cnt_sh.at[pl.ds(row0 + r * ZR, ZR)])
    plsc.subcore_barrier()

    n_my = NCHUNK // NW + jnp.where(wid < NCHUNK % NW, 1, 0)

    def chunk(i, carry):
        base = (wid + i * NW) * CH
        pltpu.sync_copy(dst_hbm.at[pl.ds(base, CH)], didx)
        pltpu.sync_copy(ones_v, cnt_sh.at[didx], add=True)
        return carry

    lax.fori_loop(0, n_my, chunk, 0)
    plsc.subcore_barrier()

    obase = cid * NP + row0
    for r in range(ROWS_PER_TILE // ZR):
        pltpu.sync_copy(cnt_sh.at[pl.ds(row0 + r * ZR, ZR)], zstage)
        pltpu.sync_copy(zstage, cnt_out.at[pl.ds(obase + r * ZR, ZR)])


def _k1b(dst):
    f = pl.kernel(
        _k1b_body,
        out_type=jax.ShapeDtypeStruct((NC * NP, D), F32),
        mesh=_mesh(),
        scratch_types=[
            pltpu.VMEM((CH,), I32),
            pltpu.VMEM((CH, D), F32),
            pltpu.VMEM_SHARED((NP, D), F32),
            pltpu.VMEM((ZR, D), F32),
        ],
    )
    return f(dst)


def _k2_body(x_ref, agg_ref, cnt_ref, wlT, wrT, w1aT, w1bT, blr, b1r,
             a_out, b_out):
    agg = agg_ref[0] + agg_ref[1]
    cnt = cnt_ref[0][:, :1] + cnt_ref[1][:, :1]
    mean = agg / jnp.maximum(cnt, 1.0)
    h = jnp.tanh(jnp.dot(mean, wlT[...], preferred_element_type=F32) + blr[...]
                 + jnp.dot(x_ref[...], wrT[...], preferred_element_type=F32))
    a_out[...] = jnp.dot(h, w1aT[...], preferred_element_type=F32)
    b_out[...] = jnp.dot(h, w1bT[...], preferred_element_type=F32) + b1r[...]


def _k2(x, aggP, cntP, W_l, b_l, W_r, W1, b1):
    BN = 1000
    full = pl.BlockSpec((D, D), lambda i: (0, 0))
    return pl.pallas_call(
        _k2_body,
        grid=(N // BN,),
        in_specs=[
            pl.BlockSpec((BN, D), lambda i: (i, 0)),
            pl.BlockSpec((NC, BN, D), lambda i: (0, i, 0)),
            pl.BlockSpec((NC, BN, D), lambda i: (0, i, 0)),
            full, full, full, full,
            pl.BlockSpec((1, D), lambda i: (0, 0)),
            pl.BlockSpec((1, D), lambda i: (0, 0)),
        ],
        out_specs=[pl.BlockSpec((BN, D), lambda i: (i, 0)),
                   pl.BlockSpec((BN, D), lambda i: (i, 0))],
        out_shape=[jax.ShapeDtypeStruct((N, D), F32),
                   jax.ShapeDtypeStruct((N, D), F32)],
    )(x, aggP, cntP, W_l.T, W_r.T, W1[:, :D].T, W1[:, D:].T,
      b_l[None, :], b1[None, :])


def _k3_body(src_hbm, dst_hbm, a_hbm, b_hbm, sa_out, sb_out,
             sidx, didx, buf_a, buf_b, sem):
    cid = lax.axis_index("c")
    sid = lax.axis_index("s")
    wid = _worker(cid, sid)
    n_my = NCHUNK // NW + jnp.where(wid < NCHUNK % NW, 1, 0)

    def chunk(i, carry):
        base = (wid + i * NW) * CH
        pltpu.sync_copy(src_hbm.at[pl.ds(base, CH)], sidx)
        pltpu.sync_copy(dst_hbm.at[pl.ds(base, CH)], didx)
        cp_a = pltpu.async_copy(a_hbm.at[sidx], buf_a, sem)
        cp_b = pltpu.async_copy(b_hbm.at[didx], buf_b, sem)
        cp_a.wait()
        cp_b.wait()
        pltpu.sync_copy(buf_a, sa_out.at[pl.ds(base, CH)])
        pltpu.sync_copy(buf_b, sb_out.at[pl.ds(base, CH)])
        return carry

    lax.fori_loop(0, n_my, chunk, 0)


def _k3(src, dst, A, B):
    f = pl.kernel(
        _k3_body,
        out_type=(jax.ShapeDtypeStruct((E, D), F32),
                  jax.ShapeDtypeStruct((E, D), F32)),
        mesh=_mesh(),
        scratch_types=[
            pltpu.VMEM((CH,), I32),
            pltpu.VMEM((CH,), I32),
            pltpu.VMEM((CH, D), F32),
            pltpu.VMEM((CH, D), F32),
            pltpu.SemaphoreType.DMA,
        ],
    )
    return f(src, dst, A, B)


def _k4_body(sa_ref, sb_ref, w2_ref, b2_ref, o_ref):
    t = jnp.tanh(sa_ref[...] + sb_ref[...])
    z = jnp.dot(t, w2_ref[...], preferred_element_type=F32) + b2_ref[...]
    o_ref[...] = jax.nn.sigmoid(jnp.tanh(z))


def _k4(sa, sb, w2col, b2):
    BE = 8000
    return pl.pallas_call(
        _k4_body,
        grid=(E // BE,),
        in_specs=[
            pl.BlockSpec((BE, D), lambda i: (i, 0)),
            pl.BlockSpec((BE, D), lambda i: (i, 0)),
            pl.BlockSpec((D, 1), lambda i: (0, 0)),
            pl.BlockSpec((1, 1), lambda i: (0, 0)),
        ],
        out_specs=pl.BlockSpec((BE, 1), lambda i: (i, 0)),
        out_shape=jax.ShapeDtypeStruct((E, 1), F32),
    )(sa, sb, w2col, b2)


def kernel(x, edge_index, W_l, b_l, W_r, W1, b1, W2, b2):
    src = edge_index[0].astype(I32)
    dst = edge_index[1].astype(I32)
    agg_flat = _k1(src, dst, x)
    cnt_flat = _k1b(dst)
    aggP = agg_flat.reshape(NC, NP, D)
    cntP = cnt_flat.reshape(NC, NP, D)
    A, B = _k2(x, aggP, cntP, W_l, b_l, W_r, W1, b1)
    sa, sb = _k3(src, dst, A, B)
    return _k4(sa, sb, W2[0][:, None], b2[None, :])
